# baseline (speedup 1.0000x reference)
import functools

import jax
import jax.numpy as jnp
from jax import lax
from jax.experimental import pallas as pl
from jax.experimental.pallas import tpu as pltpu
from jax.experimental.pallas import tpu_sc as plsc

_N = 131072
_K = 512
_D = 32
_BETA = 0.25
_BLOCK = 8192

_NC = 2
_NS = 16
_NW = _NC * _NS
_PER_W = _N // _NW
_CHUNK = 2048


def _norm_tree_lanes(x):
    t = x * x
    l = ((t[:, 0:8] + t[:, 8:16]) + t[:, 16:24]) + t[:, 24:32]
    u = l[:, 0:4] + l[:, 4:8]
    m = u[:, 0:2] + u[:, 2:4]
    return m[:, 0:1] + m[:, 1:2]


def _norm_tree_sublanes(x):
    t = x * x
    l = ((t[0:8, :] + t[8:16, :]) + t[16:24, :]) + t[24:32, :]
    u = l[0:4, :] + l[4:8, :]
    m = u[0:2, :] + u[2:4, :]
    return m[0:1, :] + m[1:2, :]


def _vq_block(zt_ref, e_ref, idx_ref, loss_ref):
    zt = zt_ref[...]
    e = e_ref[...]
    zsq = _norm_tree_sublanes(zt)
    esq = _norm_tree_lanes(e)
    e2 = e + e
    mm2 = jax.lax.dot_general(e2, zt, (((1,), (0,)), ((), ())),
                              preferred_element_type=jnp.float32)
    dist = (zsq + esq) - mm2
    iota_k = jax.lax.broadcasted_iota(jnp.int32, dist.shape, 0)
    min_d = jnp.min(dist, axis=0, keepdims=True)
    sel = dist == min_d
    idx = jnp.min(jnp.where(sel, iota_k, _K), axis=0)
    idx_ref[0, 0, :] = idx.astype(jnp.int32)
    part = jnp.sum(min_d).reshape(1, 1)

    @pl.when(pl.program_id(0) == 0)
    def _init():
        loss_ref[...] = jnp.zeros((1, 1), jnp.float32)

    loss_ref[...] += part


def _sc_gather(idx_hbm, table_hbm, out_hbm, idx_v, rows_v, sem):
    wid = lax.axis_index("s") * _NC + lax.axis_index("c")
    for c in range(_PER_W // _CHUNK):
        base = wid * _PER_W + c * _CHUNK
        pltpu.sync_copy(idx_hbm.at[pl.ds(base, _CHUNK)], idx_v)
        pltpu.async_copy(table_hbm.at[idx_v], rows_v, sem).wait()
        pltpu.sync_copy(rows_v, out_hbm.at[pl.ds(base, _CHUNK)])


_NSLICE = 4
_NH = _N // _NSLICE
_PER_W_H = _NH // _NW
_CH = min(_CHUNK, _PER_W_H)


def _sc_gather_half(idx_hbm, table_hbm, out_hbm, idx_v, rows_v, sem):
    wid = lax.axis_index("s") * _NC + lax.axis_index("c")
    for c in range(_PER_W_H // _CH):
        base = wid * _PER_W_H + c * _CH
        pltpu.sync_copy(idx_hbm.at[pl.ds(base, _CH)], idx_v)
        pltpu.async_copy(table_hbm.at[idx_v], rows_v, sem).wait()
        pltpu.sync_copy(rows_v, out_hbm.at[pl.ds(base, _CH)])


def kernel(z_e, embeddings):
    zt = z_e.T
    nb_h = _NH // _BLOCK
    gather = functools.partial(
        pl.kernel,
        mesh=plsc.VectorSubcoreMesh(core_axis_name="c", subcore_axis_name="s"),
        compiler_params=pltpu.CompilerParams(use_tc_tiling_on_sc=False),
        out_type=jax.ShapeDtypeStruct((_NH, _D), jnp.float32),
        scratch_types=[
            pltpu.VMEM((_CH,), jnp.int32),
            pltpu.VMEM((_CH, _D), jnp.float32),
            pltpu.SemaphoreType.DMA,
        ],
    )(_sc_gather_half)

    idx_halves, zq_halves, loss_sums = [], [], []
    for h in range(_NSLICE):
        off = h * nb_h
        idx3, loss_sum = pl.pallas_call(
            _vq_block,
            grid=(nb_h,),
            in_specs=[
                pl.BlockSpec((_D, _BLOCK), lambda i, off=off: (0, off + i)),
                pl.BlockSpec((_K, _D), lambda i: (0, 0)),
            ],
            out_specs=[
                pl.BlockSpec((1, 1, _BLOCK), lambda i: (i, 0, 0)),
                pl.BlockSpec((1, 1), lambda i: (0, 0)),
            ],
            out_shape=[
                jax.ShapeDtypeStruct((nb_h, 1, _BLOCK), jnp.int32),
                jax.ShapeDtypeStruct((1, 1), jnp.float32),
            ],
        )(zt, embeddings)
        idx_h = idx3.reshape(_NH)
        idx_halves.append(idx_h)
        zq_halves.append(gather(idx_h, embeddings))
        loss_sums.append(loss_sum[0, 0])

    encoding_inds = jnp.concatenate(idx_halves)
    zq = jnp.concatenate(zq_halves, axis=0)
    loss = sum(loss_sums) * ((1.0 + _BETA) / (_N * _D))
    return (zq, encoding_inds, loss)


# speedup vs baseline: 1.1603x; 1.1603x over previous
import functools

import jax
import jax.numpy as jnp
from jax import lax
from jax.experimental import pallas as pl
from jax.experimental.pallas import tpu as pltpu
from jax.experimental.pallas import tpu_sc as plsc

_N = 131072
_K = 512
_D = 32
_BETA = 0.25
_BLOCK = 8192

_NC = 2
_NS = 16
_NW = _NC * _NS
_PER_W = _N // _NW
_CHUNK = 2048


def _norm_tree_lanes(x):
    t = x * x
    l = ((t[:, 0:8] + t[:, 8:16]) + t[:, 16:24]) + t[:, 24:32]
    u = l[:, 0:4] + l[:, 4:8]
    m = u[:, 0:2] + u[:, 2:4]
    return m[:, 0:1] + m[:, 1:2]


def _norm_tree_sublanes(x):
    t = x * x
    l = ((t[0:8, :] + t[8:16, :]) + t[16:24, :]) + t[24:32, :]
    u = l[0:4, :] + l[4:8, :]
    m = u[0:2, :] + u[2:4, :]
    return m[0:1, :] + m[1:2, :]


def _vq_block(zt_ref, e_ref, idx_ref, loss_ref):
    zt = zt_ref[...]
    e = e_ref[...]
    zsq = _norm_tree_sublanes(zt)
    esq = _norm_tree_lanes(e)
    e2 = e + e
    mm2 = jax.lax.dot_general(e2, zt, (((1,), (0,)), ((), ())),
                              preferred_element_type=jnp.float32)
    dist = (zsq + esq) - mm2
    iota_k = jax.lax.broadcasted_iota(jnp.int32, dist.shape, 0)
    min_d = jnp.min(dist, axis=0, keepdims=True)
    sel = dist == min_d
    idx = jnp.min(jnp.where(sel, iota_k, _K), axis=0)
    idx_ref[0, 0, :] = idx.astype(jnp.int32)
    part = jnp.sum(min_d).reshape(1, 1)

    @pl.when(pl.program_id(0) == 0)
    def _init():
        loss_ref[...] = jnp.zeros((1, 1), jnp.float32)

    loss_ref[...] += part


def _sc_gather(idx_hbm, table_hbm, out_hbm, idx_v, rows_v, sem):
    wid = lax.axis_index("s") * _NC + lax.axis_index("c")
    for c in range(_PER_W // _CHUNK):
        base = wid * _PER_W + c * _CHUNK
        pltpu.sync_copy(idx_hbm.at[pl.ds(base, _CHUNK)], idx_v)
        pltpu.async_copy(table_hbm.at[idx_v], rows_v, sem).wait()
        pltpu.sync_copy(rows_v, out_hbm.at[pl.ds(base, _CHUNK)])


_NSLICE = 2
_NH = _N // _NSLICE
_PER_W_H = _NH // _NW
_CH = min(_CHUNK, _PER_W_H)


def _sc_gather_half(idx_hbm, table_hbm, out_hbm, idx_v, rows_v, sem):
    wid = lax.axis_index("s") * _NC + lax.axis_index("c")
    for c in range(_PER_W_H // _CH):
        base = wid * _PER_W_H + c * _CH
        pltpu.sync_copy(idx_hbm.at[pl.ds(base, _CH)], idx_v)
        pltpu.async_copy(table_hbm.at[idx_v], rows_v, sem).wait()
        pltpu.sync_copy(rows_v, out_hbm.at[pl.ds(base, _CH)])


def kernel(z_e, embeddings):
    zt = z_e.T
    nb_h = _NH // _BLOCK
    gather = functools.partial(
        pl.kernel,
        mesh=plsc.VectorSubcoreMesh(core_axis_name="c", subcore_axis_name="s"),
        compiler_params=pltpu.CompilerParams(use_tc_tiling_on_sc=False),
        out_type=jax.ShapeDtypeStruct((_NH, _D), jnp.float32),
        scratch_types=[
            pltpu.VMEM((_CH,), jnp.int32),
            pltpu.VMEM((_CH, _D), jnp.float32),
            pltpu.SemaphoreType.DMA,
        ],
    )(_sc_gather_half)

    idx_halves, zq_halves, loss_sums = [], [], []
    for h in range(_NSLICE):
        off = h * nb_h
        idx3, loss_sum = pl.pallas_call(
            _vq_block,
            grid=(nb_h,),
            in_specs=[
                pl.BlockSpec((_D, _BLOCK), lambda i, off=off: (0, off + i)),
                pl.BlockSpec((_K, _D), lambda i: (0, 0)),
            ],
            out_specs=[
                pl.BlockSpec((1, 1, _BLOCK), lambda i: (i, 0, 0)),
                pl.BlockSpec((1, 1), lambda i: (0, 0)),
            ],
            out_shape=[
                jax.ShapeDtypeStruct((nb_h, 1, _BLOCK), jnp.int32),
                jax.ShapeDtypeStruct((1, 1), jnp.float32),
            ],
        )(zt, embeddings)
        idx_h = idx3.reshape(_NH)
        idx_halves.append(idx_h)
        zq_halves.append(gather(idx_h, embeddings))
        loss_sums.append(loss_sum[0, 0])

    encoding_inds = jnp.concatenate(idx_halves)
    zq = jnp.concatenate(zq_halves, axis=0)
    loss = sum(loss_sums) * ((1.0 + _BETA) / (_N * _D))
    return (zq, encoding_inds, loss)
